# baseline (speedup 1.0000x reference)
import jax
import jax.numpy as jnp
from jax.experimental import pallas as pl
from jax.experimental.pallas import tpu as pltpu


def _fused_kernel(x_ref, w_ref, b_ref, p_ref, o_ref, acc_ref):
    k = pl.program_id(0)
    nk = pl.num_programs(0)
    i = pl.program_id(1)
    bm = x_ref.shape[0]
    part = jnp.dot(x_ref[...], w_ref[...], preferred_element_type=jnp.float32)
    rows = pl.ds(i * bm, bm)

    @pl.when(k == 0)
    def _init():
        acc_ref[rows, :] = part + b_ref[...]

    @pl.when(k != 0)
    def _accum():
        acc_ref[rows, :] += part

    @pl.when(k == nk - 1)
    def _epilogue():
        f = (acc_ref[rows, :] > 0.5).astype(jnp.float32)
        fp = jax.lax.dot_general(
            f, p_ref[...], (((1,), (1,)), ((), ())),
            preferred_element_type=jnp.float32)
        o_ref[...] = fp - jnp.sum(f, axis=1, keepdims=True)


def kernel(x, W, b, predicate_matrix):
    bsz, d_in = x.shape
    nf = W.shape[1]
    nc = predicate_matrix.shape[0]
    bm = 256
    bk = 1024
    b2 = b.reshape(1, nf)
    return pl.pallas_call(
        _fused_kernel,
        grid=(d_in // bk, bsz // bm),
        in_specs=[
            pl.BlockSpec((bm, bk), lambda k, i: (i, k)),
            pl.BlockSpec((bk, nf), lambda k, i: (k, 0)),
            pl.BlockSpec((1, nf), lambda k, i: (0, 0)),
            pl.BlockSpec((nc, nf), lambda k, i: (0, 0)),
        ],
        out_specs=pl.BlockSpec((bm, nc), lambda k, i: (i, 0)),
        out_shape=jax.ShapeDtypeStruct((bsz, nc), jnp.float32),
        scratch_shapes=[pltpu.VMEM((bsz, nf), jnp.float32)],
    )(x, W, b2, predicate_matrix)


# speedup vs baseline: 1.4069x; 1.4069x over previous
import jax
import jax.numpy as jnp
from jax.experimental import pallas as pl


def _fused_kernel(x_ref, w_ref, b_ref, p_ref, o_ref):
    feats = jnp.dot(x_ref[...], w_ref[...], preferred_element_type=jnp.float32)
    feats = feats + b_ref[...]
    f = (feats > 0.5).astype(jnp.float32)
    fp = jax.lax.dot_general(
        f, p_ref[...], (((1,), (1,)), ((), ())),
        preferred_element_type=jnp.float32)
    o_ref[...] = fp - jnp.sum(f, axis=1, keepdims=True)


def kernel(x, W, b, predicate_matrix):
    bsz, d_in = x.shape
    nf = W.shape[1]
    nc = predicate_matrix.shape[0]
    bm = 128
    b2 = b.reshape(1, nf)
    return pl.pallas_call(
        _fused_kernel,
        grid=(bsz // bm,),
        in_specs=[
            pl.BlockSpec((bm, d_in), lambda i: (i, 0)),
            pl.BlockSpec((d_in, nf), lambda i: (0, 0)),
            pl.BlockSpec((1, nf), lambda i: (0, 0)),
            pl.BlockSpec((nc, nf), lambda i: (0, 0)),
        ],
        out_specs=pl.BlockSpec((bm, nc), lambda i: (i, 0)),
        out_shape=jax.ShapeDtypeStruct((bsz, nc), jnp.float32),
    )(x, W, b2, predicate_matrix)


# speedup vs baseline: 1.6610x; 1.1806x over previous
import jax
import jax.numpy as jnp
from jax.experimental import pallas as pl


def _fused_kernel(x_ref, w_ref, b_ref, p_ref, o_ref):
    feats = jnp.dot(x_ref[...], w_ref[...], preferred_element_type=jnp.float32)
    feats = feats + b_ref[...]
    f = (feats > 0.5).astype(jnp.float32)
    fp = jax.lax.dot_general(
        f, p_ref[...], (((1,), (1,)), ((), ())),
        preferred_element_type=jnp.float32)
    o_ref[...] = fp - jnp.sum(f, axis=1, keepdims=True)


def kernel(x, W, b, predicate_matrix):
    bsz, d_in = x.shape
    nf = W.shape[1]
    nc = predicate_matrix.shape[0]
    bm = 512
    b2 = b.reshape(1, nf)
    return pl.pallas_call(
        _fused_kernel,
        grid=(bsz // bm,),
        in_specs=[
            pl.BlockSpec((bm, d_in), lambda i: (i, 0)),
            pl.BlockSpec((d_in, nf), lambda i: (0, 0)),
            pl.BlockSpec((1, nf), lambda i: (0, 0)),
            pl.BlockSpec((nc, nf), lambda i: (0, 0)),
        ],
        out_specs=pl.BlockSpec((bm, nc), lambda i: (i, 0)),
        out_shape=jax.ShapeDtypeStruct((bsz, nc), jnp.float32),
    )(x, W, b2, predicate_matrix)
